# baseline (speedup 1.0000x reference)
import functools

import jax
import jax.numpy as jnp
import numpy as np
from jax import lax
from jax.experimental import pallas as pl
from jax.experimental.pallas import tpu as pltpu
from jax.experimental.pallas import tpu_sc as plsc

N_BINS = 256
HIDDEN = 256
PITCH_MIN = np.float32(50.0)
INV_RANGE = np.float32(1.0) / np.float32(350.0)

NC, NS, LANES = 2, 16, 16
NW = NC * NS

B = 4096 * 200
B_PER_W = B // NW
CHUNK = 128
N_CHUNKS = B_PER_W // CHUNK
GROUPS = CHUNK // LANES


def _bins(pitch_c, bases_c):
    def bins_group(i, carry):
        p = pitch_c[pl.ds(i * LANES, LANES)]
        u = (p - PITCH_MIN) * INV_RANGE
        v = jnp.clip(u, 0.0, 1.0)
        b = (v * np.float32(N_BINS - 1)).astype(jnp.int32)
        bases_c[pl.ds(i * LANES, LANES)] = b * HIDDEN
        return carry

    lax.fori_loop(0, GROUPS, bins_group, 0)


def _body(pitch_hbm, table_hbm, out_hbm, table_v, pitch_c, bases_c, sem):
    wid = lax.axis_index("s") * NC + lax.axis_index("c")
    base = wid * B_PER_W
    sem_p, sem_w = sem

    pltpu.sync_copy(table_hbm, table_v)

    def start_pitch(j, buf):
        pltpu.make_async_copy(
            pitch_hbm.at[pl.ds(base + j * CHUNK, CHUNK)], buf, sem_p
        ).start()

    def wait_pitch():
        pltpu.make_async_copy(
            pitch_hbm.at[pl.ds(base, CHUNK)], pitch_c[0], sem_p
        ).wait()

    def wait_chunk():
        pltpu.make_async_copy(
            table_v.at[pl.ds(0, CHUNK * HIDDEN)],
            out_hbm.at[pl.ds(base * HIDDEN, CHUNK * HIDDEN)],
            sem_w,
        ).wait()

    start_pitch(0, pitch_c[0])
    start_pitch(1, pitch_c[1])

    def chunk_step(j, b):
        wait_pitch()
        _bins(pitch_c[b], bases_c)

        @pl.when(j + 2 < N_CHUNKS)
        def _():
            start_pitch(j + 2, pitch_c[b])

        @pl.when(j >= 1)
        def _():
            wait_chunk()

        def fire_group(g, carry):
            goff = g * LANES
            bvec = bases_c[pl.ds(goff, LANES)]
            row0 = pl.multiple_of((base + j * CHUNK + goff) * HIDDEN, HIDDEN)
            for r in range(0, LANES, 2):
                s = pl.multiple_of(bvec[r], HIDDEN)
                pltpu.make_async_copy(
                    table_v.at[pl.ds(s, 2 * HIDDEN)],
                    out_hbm.at[pl.ds(row0 + r * HIDDEN, 2 * HIDDEN)],
                    sem_w,
                ).start()
            return carry

        lax.fori_loop(0, GROUPS, fire_group, 0)

    def outer(jo, carry):
        for b in range(2):
            chunk_step(jo * 2 + b, b)
        return carry

    lax.fori_loop(0, N_CHUNKS // 2, outer, 0)
    wait_chunk()


@functools.partial(
    pl.kernel,
    out_type=jax.ShapeDtypeStruct((B * HIDDEN,), jnp.float32),
    mesh=plsc.VectorSubcoreMesh(core_axis_name="c", subcore_axis_name="s"),
    compiler_params=pltpu.CompilerParams(needs_layout_passes=False),
    scratch_types=[
        pltpu.VMEM((N_BINS * HIDDEN,), jnp.float32),
        (pltpu.VMEM((CHUNK,), jnp.float32), pltpu.VMEM((CHUNK,), jnp.float32)),
        pltpu.VMEM((CHUNK,), jnp.int32),
        (pltpu.SemaphoreType.DMA, pltpu.SemaphoreType.DMA),
    ],
)
def _pitch_embed(pitch_hbm, table_hbm, out_hbm, table_v, pitch_c, bases_c, sem):
    _body(pitch_hbm, table_hbm, out_hbm, table_v, pitch_c, bases_c, sem)


def kernel(pitch, table):
    out = _pitch_embed(pitch.reshape(-1), table.reshape(-1))
    return out.reshape(*pitch.shape, HIDDEN)
